# baseline (speedup 1.0000x reference)
import functools

import jax
import jax.numpy as jnp
from jax import lax
from jax.experimental import pallas as pl
from jax.experimental.pallas import tpu as pltpu
from jax.experimental.pallas import tpu_sc as plsc

N = 10000
E = 160000
D = 256
H = 256
OUT = 10
EPS = 1e-5

NS = 16
W = 128
E_PAD = 163840
NWIN = E_PAD // W
G = 16
WPT = NWIN // NS
N_PAD = 10240
A_ROWS = 10112


def _make_seg_sum(compute_deg: bool):
    out_type = [jax.ShapeDtypeStruct((N, D), jnp.float32)]
    if compute_deg:
        out_type.append(jax.ShapeDtypeStruct((N_PAD,), jnp.float32))
    scratch = [
        pltpu.VMEM((2, G, W), jnp.int32),
        pltpu.VMEM((2, G, W), jnp.int32),
        pltpu.VMEM((2, W, 128), jnp.float32),
        pltpu.VMEM((W,), jnp.float32),
        pltpu.VMEM((640,), jnp.float32),
        pltpu.VMEM_SHARED((A_ROWS, 128), jnp.float32),
        pltpu.VMEM_SHARED((N_PAD,), jnp.float32),
    ] + [pltpu.SemaphoreType.DMA] * 6
    mesh = plsc.VectorSubcoreMesh(core_axis_name="c", subcore_axis_name="s")

    @functools.partial(pl.kernel, mesh=mesh, out_type=out_type,
                       scratch_types=scratch)
    def seg_sum(x_hbm, idx_hbm, dst_hbm, *refs):
        if compute_deg:
            summed_out, deg_out = refs[0], refs[1]
            refs = refs[2:]
        else:
            summed_out = refs[0]
            refs = refs[1:]
        (gidx_v, didx_v, rows_v, ones_v, z1d_v, acc_sh, deg_sh,
         sga, sgb, st0, st1, si, sd) = refs
        st = [st0, st1]
        c = lax.axis_index("c")
        s = lax.axis_index("s")

        def zrow(i, carry):
            for j in range(8):
                rows_v[0, i, pl.ds(j * 16, 16)] = jnp.zeros((16,), jnp.float32)
            return carry
        lax.fori_loop(0, W, zrow, 0)
        span = WPT * s
        ZCH = A_ROWS // W
        def zchunk(j, carry):
            ch = s + NS * j

            @pl.when(ch < ZCH)
            def _():
                pltpu.sync_copy(rows_v.at[0], acc_sh.at[pl.ds(W * ch, W)])
            return carry
        lax.fori_loop(0, (ZCH + NS - 1) // NS, zchunk, 0)
        if compute_deg:
            for j in range(W // 16):
                ones_v[pl.ds(j * 16, 16)] = jnp.full((16,), 1.0, jnp.float32)
            for j in range(40):
                z1d_v[pl.ds(j * 16, 16)] = jnp.zeros((16,), jnp.float32)

            @pl.when(c == 0)
            def _():
                pltpu.sync_copy(z1d_v, deg_sh.at[pl.ds(s * 640, 640)])
        plsc.subcore_barrier()

        cofs = c * 128

        def gather(idx_row, b, sem):
            pltpu.async_copy(x_hbm.at[idx_row, pl.ds(cofs, 128)],
                             rows_v.at[b], sem)

        def wait_g(sem, b):
            pltpu.make_async_copy(x_hbm.at[gidx_v.at[0, 0], pl.ds(cofs, 128)],
                                  rows_v.at[b], sem).wait()

        def wait_s(b):
            pltpu.make_async_copy(x_hbm.at[gidx_v.at[0, 0], pl.ds(cofs, 128)],
                                  rows_v.at[b], st[b]).wait()

        def wait_d():
            pltpu.make_async_copy(ones_v, deg_sh.at[didx_v.at[0, 0]],
                                  sd).wait()

        pltpu.sync_copy(idx_hbm.at[pl.ds(span, G)], gidx_v.at[0])
        pltpu.sync_copy(dst_hbm.at[pl.ds(span, G)], didx_v.at[0])
        gather(gidx_v.at[0, 0], 0, sga)
        gather(gidx_v.at[0, 1], 1, sgb)

        NG = WPT // G

        def group_body(g, carry):
            slot = g % 2
            nslot = (g + 1) % 2
            for k in range(G):
                b = k % 2
                sgp = sga if k % 2 == 0 else sgb
                wait_g(sgp, b)
                pltpu.async_copy(rows_v.at[b], acc_sh.at[didx_v.at[slot, k]],
                                 st[b], add=True)
                if k < G - 2:
                    wait_s(b)
                    gather(gidx_v.at[slot, k + 2], b, sgp)
                else:
                    if k == G - 2:
                        @pl.when(g < NG - 1)
                        def _():
                            pltpu.make_async_copy(
                                idx_hbm.at[pl.ds(span, G)],
                                gidx_v.at[0], si).wait()
                            pltpu.make_async_copy(
                                dst_hbm.at[pl.ds(span, G)],
                                didx_v.at[0], si).wait()

                    @pl.when(g < NG - 1)
                    def _(k=k, b=b, sgp=sgp, nslot=nslot):
                        wait_s(b)
                        gather(gidx_v.at[nslot, k + 2 - G], b, sgp)
                if compute_deg:
                    @pl.when(c == 0)
                    def _(k=k, slot=slot):
                        if k == 0:
                            @pl.when(g > 0)
                            def _():
                                wait_d()
                        else:
                            wait_d()
                        pltpu.async_copy(ones_v,
                                         deg_sh.at[didx_v.at[slot, k]],
                                         sd, add=True)
                if k == 2:
                    @pl.when(g < NG - 1)
                    def _(nslot=nslot):
                        nbase = span + G * (g + 1)
                        pltpu.async_copy(idx_hbm.at[pl.ds(nbase, G)],
                                         gidx_v.at[nslot], si)
                        pltpu.async_copy(dst_hbm.at[pl.ds(nbase, G)],
                                         didx_v.at[nslot], si)
            return carry
        lax.fori_loop(0, NG, group_body, 0)

        for b in range(2):
            wait_s(b)
        if compute_deg:
            @pl.when(c == 0)
            def _():
                wait_d()
        plsc.subcore_barrier()

        for cc in range(2):
            @pl.when(c == cc)
            def _(cc=cc):
                def wchunk(j, carry):
                    ch = s + NS * j

                    @pl.when(ch < 78)
                    def _():
                        pltpu.sync_copy(
                            acc_sh.at[pl.ds(128 * ch, 128)],
                            summed_out.at[pl.ds(128 * ch, 128),
                                          pl.ds(cc * 128, 128)])
                    return carry
                lax.fori_loop(0, 5, wchunk, 0)

                @pl.when(s == 15)
                def _():
                    pltpu.sync_copy(
                        acc_sh.at[pl.ds(9984, 16)],
                        summed_out.at[pl.ds(9984, 16),
                                      pl.ds(cc * 128, 128)])
        if compute_deg:
            @pl.when(c == 0)
            def _():
                pltpu.sync_copy(deg_sh.at[pl.ds(s * 640, 640)],
                                deg_out.at[pl.ds(s * 640, 640)])

    return seg_sum


_seg_sum_deg = _make_seg_sum(True)
_seg_sum = _make_seg_sum(False)

BR = 2000


def _dense0_body(s_ref, d_ref, x_ref, wl_ref, wr_ref, c_ref, o_ref):
    inv = 1.0 / jnp.maximum(d_ref[...], 1.0)
    z = jnp.dot(s_ref[...] * inv, wl_ref[...],
                preferred_element_type=jnp.float32)
    z = z + jnp.dot(x_ref[...], wr_ref[...],
                    preferred_element_type=jnp.float32)
    o_ref[...] = jnp.maximum(z + c_ref[...], 0.0)


_dense0 = pl.pallas_call(
    _dense0_body,
    grid=(N // BR,),
    in_specs=[
        pl.BlockSpec((BR, D), lambda i: (i, 0)),
        pl.BlockSpec((BR, 1), lambda i: (i, 0)),
        pl.BlockSpec((BR, D), lambda i: (i, 0)),
        pl.BlockSpec((D, H), lambda i: (0, 0)),
        pl.BlockSpec((D, H), lambda i: (0, 0)),
        pl.BlockSpec((1, H), lambda i: (0, 0)),
    ],
    out_specs=pl.BlockSpec((BR, H), lambda i: (i, 0)),
    out_shape=jax.ShapeDtypeStruct((N, H), jnp.float32),
)


def _dense1_body(s_ref, d_ref, h_ref, wl_ref, wr_ref, c_ref,
                 w1_ref, b1_ref, w2_ref, b2_ref, o_ref):
    inv = 1.0 / jnp.maximum(d_ref[...], 1.0)
    z = jnp.dot(s_ref[...] * inv, wl_ref[...],
                preferred_element_type=jnp.float32)
    z = z + jnp.dot(h_ref[...], wr_ref[...],
                    preferred_element_type=jnp.float32)
    h2 = jnp.maximum(z + c_ref[...], 0.0)
    t = jnp.maximum(jnp.dot(h2, w1_ref[...],
                            preferred_element_type=jnp.float32) + b1_ref[...],
                    0.0)
    o_ref[...] = jnp.dot(t, w2_ref[...],
                         preferred_element_type=jnp.float32) + b2_ref[...]


_dense1 = pl.pallas_call(
    _dense1_body,
    grid=(N // BR,),
    in_specs=[
        pl.BlockSpec((BR, H), lambda i: (i, 0)),
        pl.BlockSpec((BR, 1), lambda i: (i, 0)),
        pl.BlockSpec((BR, H), lambda i: (i, 0)),
        pl.BlockSpec((H, H), lambda i: (0, 0)),
        pl.BlockSpec((H, H), lambda i: (0, 0)),
        pl.BlockSpec((1, H), lambda i: (0, 0)),
        pl.BlockSpec((H, H // 2), lambda i: (0, 0)),
        pl.BlockSpec((1, H // 2), lambda i: (0, 0)),
        pl.BlockSpec((H // 2, OUT), lambda i: (0, 0)),
        pl.BlockSpec((1, OUT), lambda i: (0, 0)),
    ],
    out_specs=pl.BlockSpec((BR, OUT), lambda i: (i, 0)),
    out_shape=jax.ShapeDtypeStruct((N, OUT), jnp.float32),
)


def kernel(x, edge_index, Wl0, bl0, Wr0, gamma0, beta0, rm0, rv0,
           Wl1, bl1, Wr1, gamma1, beta1, rm1, rv1,
           Wc1, bc1, Wc2, bc2):
    src = edge_index[0]
    dst = edge_index[1]
    pe = jnp.arange(E_PAD - E, dtype=jnp.int32)
    srcw = jnp.concatenate([src, (pe * 2503) % N]).reshape(NWIN, W)
    dstw = jnp.concatenate([dst, N + pe % (A_ROWS - N)]).reshape(NWIN, W)

    g0 = gamma0 * lax.rsqrt(rv0 + EPS)
    WlT0 = Wl0.T * g0[None, :]
    WrT0 = Wr0.T * g0[None, :]
    c0 = ((bl0 - rm0) * g0 + beta0).reshape(1, H)
    g1 = gamma1 * lax.rsqrt(rv1 + EPS)
    WlT1 = Wl1.T * g1[None, :]
    WrT1 = Wr1.T * g1[None, :]
    c1 = ((bl1 - rm1) * g1 + beta1).reshape(1, H)

    summed0, deg = _seg_sum_deg(x, srcw, dstw)
    degc = deg.reshape(N_PAD, 1)
    h1 = _dense0(summed0, degc, x, WlT0, WrT0, c0)
    summed1 = _seg_sum(h1, srcw, dstw)[0]
    out = _dense1(summed1, degc, h1, WlT1, WrT1, c1,
                  Wc1.T, bc1.reshape(1, H // 2), Wc2.T, bc2.reshape(1, OUT))
    return out


# speedup vs baseline: 1.0096x; 1.0096x over previous
import functools

import jax
import jax.numpy as jnp
from jax import lax
from jax.experimental import pallas as pl
from jax.experimental.pallas import tpu as pltpu
from jax.experimental.pallas import tpu_sc as plsc

N = 10000
E = 160000
D = 256
H = 256
OUT = 10
EPS = 1e-5

NS = 16
W = 128
E_PAD = 163840
NWIN = E_PAD // W
G = 8
WPT = NWIN // NS
N_PAD = 10240
A_ROWS = 10112


def _make_seg_sum(compute_deg: bool):
    out_type = [jax.ShapeDtypeStruct((N, D), jnp.float32)]
    if compute_deg:
        out_type.append(jax.ShapeDtypeStruct((N_PAD,), jnp.float32))
    scratch = [
        pltpu.VMEM((2, G, W), jnp.int32),
        pltpu.VMEM((2, G, W), jnp.int32),
        pltpu.VMEM((2, W, 128), jnp.float32),
        pltpu.VMEM((W,), jnp.float32),
        pltpu.VMEM((640,), jnp.float32),
        pltpu.VMEM_SHARED((A_ROWS, 128), jnp.float32),
        pltpu.VMEM_SHARED((N_PAD,), jnp.float32),
    ] + [pltpu.SemaphoreType.DMA] * 6
    mesh = plsc.VectorSubcoreMesh(core_axis_name="c", subcore_axis_name="s")

    @functools.partial(pl.kernel, mesh=mesh, out_type=out_type,
                       scratch_types=scratch)
    def seg_sum(x_hbm, idx_hbm, dst_hbm, *refs):
        if compute_deg:
            summed_out, deg_out = refs[0], refs[1]
            refs = refs[2:]
        else:
            summed_out = refs[0]
            refs = refs[1:]
        (gidx_v, didx_v, rows_v, ones_v, z1d_v, acc_sh, deg_sh,
         sga, sgb, st0, st1, si, sd) = refs
        st = [st0, st1]
        c = lax.axis_index("c")
        s = lax.axis_index("s")

        def zrow(i, carry):
            for j in range(8):
                rows_v[0, i, pl.ds(j * 16, 16)] = jnp.zeros((16,), jnp.float32)
            return carry
        lax.fori_loop(0, W, zrow, 0)
        span = WPT * s
        ZCH = A_ROWS // W
        def zchunk(j, carry):
            ch = s + NS * j

            @pl.when(ch < ZCH)
            def _():
                pltpu.sync_copy(rows_v.at[0], acc_sh.at[pl.ds(W * ch, W)])
            return carry
        lax.fori_loop(0, (ZCH + NS - 1) // NS, zchunk, 0)
        if compute_deg:
            for j in range(W // 16):
                ones_v[pl.ds(j * 16, 16)] = jnp.full((16,), 1.0, jnp.float32)
            for j in range(40):
                z1d_v[pl.ds(j * 16, 16)] = jnp.zeros((16,), jnp.float32)

            @pl.when(c == 0)
            def _():
                pltpu.sync_copy(z1d_v, deg_sh.at[pl.ds(s * 640, 640)])
        plsc.subcore_barrier()

        cofs = c * 128

        def gather(idx_row, b, sem):
            pltpu.async_copy(x_hbm.at[idx_row, pl.ds(cofs, 128)],
                             rows_v.at[b], sem)

        def wait_g(sem, b):
            pltpu.make_async_copy(x_hbm.at[gidx_v.at[0, 0], pl.ds(cofs, 128)],
                                  rows_v.at[b], sem).wait()

        def wait_s(b):
            pltpu.make_async_copy(x_hbm.at[gidx_v.at[0, 0], pl.ds(cofs, 128)],
                                  rows_v.at[b], st[b]).wait()

        def wait_d():
            pltpu.make_async_copy(ones_v, deg_sh.at[didx_v.at[0, 0]],
                                  sd).wait()

        pltpu.sync_copy(idx_hbm.at[pl.ds(span, G)], gidx_v.at[0])
        pltpu.sync_copy(dst_hbm.at[pl.ds(span, G)], didx_v.at[0])
        gather(gidx_v.at[0, 0], 0, sga)
        gather(gidx_v.at[0, 1], 1, sgb)

        NG = WPT // G

        def group_body(g, carry):
            slot = g % 2
            nslot = (g + 1) % 2
            for k in range(G):
                b = k % 2
                sgp = sga if k % 2 == 0 else sgb
                wait_g(sgp, b)
                pltpu.async_copy(rows_v.at[b], acc_sh.at[didx_v.at[slot, k]],
                                 st[b], add=True)
                if k < G - 2:
                    wait_s(b)
                    gather(gidx_v.at[slot, k + 2], b, sgp)
                else:
                    if k == G - 2:
                        @pl.when(g < NG - 1)
                        def _():
                            pltpu.make_async_copy(
                                idx_hbm.at[pl.ds(span, G)],
                                gidx_v.at[0], si).wait()
                            pltpu.make_async_copy(
                                dst_hbm.at[pl.ds(span, G)],
                                didx_v.at[0], si).wait()

                    @pl.when(g < NG - 1)
                    def _(k=k, b=b, sgp=sgp, nslot=nslot):
                        wait_s(b)
                        gather(gidx_v.at[nslot, k + 2 - G], b, sgp)
                if compute_deg:
                    @pl.when(c == 0)
                    def _(k=k, slot=slot):
                        if k == 0:
                            @pl.when(g > 0)
                            def _():
                                wait_d()
                        else:
                            wait_d()
                        pltpu.async_copy(ones_v,
                                         deg_sh.at[didx_v.at[slot, k]],
                                         sd, add=True)
                if k == 2:
                    @pl.when(g < NG - 1)
                    def _(nslot=nslot):
                        nbase = span + G * (g + 1)
                        pltpu.async_copy(idx_hbm.at[pl.ds(nbase, G)],
                                         gidx_v.at[nslot], si)
                        pltpu.async_copy(dst_hbm.at[pl.ds(nbase, G)],
                                         didx_v.at[nslot], si)
            return carry
        lax.fori_loop(0, NG, group_body, 0)

        for b in range(2):
            wait_s(b)
        if compute_deg:
            @pl.when(c == 0)
            def _():
                wait_d()
        plsc.subcore_barrier()

        for cc in range(2):
            @pl.when(c == cc)
            def _(cc=cc):
                def wchunk(j, carry):
                    ch = s + NS * j

                    @pl.when(ch < 78)
                    def _():
                        pltpu.sync_copy(
                            acc_sh.at[pl.ds(128 * ch, 128)],
                            summed_out.at[pl.ds(128 * ch, 128),
                                          pl.ds(cc * 128, 128)])
                    return carry
                lax.fori_loop(0, 5, wchunk, 0)

                @pl.when(s == 15)
                def _():
                    pltpu.sync_copy(
                        acc_sh.at[pl.ds(9984, 16)],
                        summed_out.at[pl.ds(9984, 16),
                                      pl.ds(cc * 128, 128)])
        if compute_deg:
            @pl.when(c == 0)
            def _():
                pltpu.sync_copy(deg_sh.at[pl.ds(s * 640, 640)],
                                deg_out.at[pl.ds(s * 640, 640)])

    return seg_sum


_seg_sum_deg = _make_seg_sum(True)
_seg_sum = _make_seg_sum(False)

BR = 2000


def _dense0_body(s_ref, d_ref, x_ref, wl_ref, wr_ref, c_ref, o_ref):
    inv = 1.0 / jnp.maximum(d_ref[...], 1.0)
    z = jnp.dot(s_ref[...] * inv, wl_ref[...],
                preferred_element_type=jnp.float32)
    z = z + jnp.dot(x_ref[...], wr_ref[...],
                    preferred_element_type=jnp.float32)
    o_ref[...] = jnp.maximum(z + c_ref[...], 0.0)


_dense0 = pl.pallas_call(
    _dense0_body,
    grid=(N // BR,),
    in_specs=[
        pl.BlockSpec((BR, D), lambda i: (i, 0)),
        pl.BlockSpec((BR, 1), lambda i: (i, 0)),
        pl.BlockSpec((BR, D), lambda i: (i, 0)),
        pl.BlockSpec((D, H), lambda i: (0, 0)),
        pl.BlockSpec((D, H), lambda i: (0, 0)),
        pl.BlockSpec((1, H), lambda i: (0, 0)),
    ],
    out_specs=pl.BlockSpec((BR, H), lambda i: (i, 0)),
    out_shape=jax.ShapeDtypeStruct((N, H), jnp.float32),
)


def _dense1_body(s_ref, d_ref, h_ref, wl_ref, wr_ref, c_ref,
                 w1_ref, b1_ref, w2_ref, b2_ref, o_ref):
    inv = 1.0 / jnp.maximum(d_ref[...], 1.0)
    z = jnp.dot(s_ref[...] * inv, wl_ref[...],
                preferred_element_type=jnp.float32)
    z = z + jnp.dot(h_ref[...], wr_ref[...],
                    preferred_element_type=jnp.float32)
    h2 = jnp.maximum(z + c_ref[...], 0.0)
    t = jnp.maximum(jnp.dot(h2, w1_ref[...],
                            preferred_element_type=jnp.float32) + b1_ref[...],
                    0.0)
    o_ref[...] = jnp.dot(t, w2_ref[...],
                         preferred_element_type=jnp.float32) + b2_ref[...]


_dense1 = pl.pallas_call(
    _dense1_body,
    grid=(N // BR,),
    in_specs=[
        pl.BlockSpec((BR, H), lambda i: (i, 0)),
        pl.BlockSpec((BR, 1), lambda i: (i, 0)),
        pl.BlockSpec((BR, H), lambda i: (i, 0)),
        pl.BlockSpec((H, H), lambda i: (0, 0)),
        pl.BlockSpec((H, H), lambda i: (0, 0)),
        pl.BlockSpec((1, H), lambda i: (0, 0)),
        pl.BlockSpec((H, H // 2), lambda i: (0, 0)),
        pl.BlockSpec((1, H // 2), lambda i: (0, 0)),
        pl.BlockSpec((H // 2, OUT), lambda i: (0, 0)),
        pl.BlockSpec((1, OUT), lambda i: (0, 0)),
    ],
    out_specs=pl.BlockSpec((BR, OUT), lambda i: (i, 0)),
    out_shape=jax.ShapeDtypeStruct((N, OUT), jnp.float32),
)


def kernel(x, edge_index, Wl0, bl0, Wr0, gamma0, beta0, rm0, rv0,
           Wl1, bl1, Wr1, gamma1, beta1, rm1, rv1,
           Wc1, bc1, Wc2, bc2):
    src = edge_index[0]
    dst = edge_index[1]
    pe = jnp.arange(E_PAD - E, dtype=jnp.int32)
    srcw = jnp.concatenate([src, (pe * 2503) % N]).reshape(NWIN, W)
    dstw = jnp.concatenate([dst, N + pe % (A_ROWS - N)]).reshape(NWIN, W)

    g0 = gamma0 * lax.rsqrt(rv0 + EPS)
    WlT0 = Wl0.T * g0[None, :]
    WrT0 = Wr0.T * g0[None, :]
    c0 = ((bl0 - rm0) * g0 + beta0).reshape(1, H)
    g1 = gamma1 * lax.rsqrt(rv1 + EPS)
    WlT1 = Wl1.T * g1[None, :]
    WrT1 = Wr1.T * g1[None, :]
    c1 = ((bl1 - rm1) * g1 + beta1).reshape(1, H)

    summed0, deg = _seg_sum_deg(x, srcw, dstw)
    degc = deg[:N].reshape(N, 1)
    h1 = _dense0(summed0, degc, x, WlT0, WrT0, c0)
    summed1 = _seg_sum(h1, srcw, dstw)[0]
    out = _dense1(summed1, degc, h1, WlT1, WrT1, c1,
                  Wc1.T, bc1.reshape(1, H // 2), Wc2.T, bc2.reshape(1, OUT))
    return out
